# baseline (speedup 1.0000x reference)
import functools

import jax
import jax.numpy as jnp
from jax import lax
from jax.experimental import pallas as pl
from jax.experimental.pallas import tpu as pltpu
from jax.experimental.pallas import tpu_sc as plsc

B = 16384
D = 16
L = 16
NC = 2
NS = 16
NW = NC * NS
BPW = B // NW
NCHUNK = 4
CHUNK = BPW // NCHUNK
GPC = CHUNK // L

NU = 1000000
NSK = 100000
S = 512
SR = S // 8
NFULL_U = NU // S
NFULL_S = NSK // S
TU = (NFULL_U + NW - 1) // NW
TS = (NFULL_S + NW - 1) // NW


def _xperm(x, s, lane):
    return jnp.take_along_axis(x, lane ^ s, axis=0, mode="promise_in_bounds")


def _transpose16(vecs, lane):
    for s in (8, 4, 2, 1):
        m = (lane & s) == 0
        nv = list(vecs)
        for i in range(L):
            if i & s == 0:
                a, b = vecs[i], vecs[i | s]
                nv[i] = jnp.where(m, a, _xperm(b, s, lane))
                nv[i | s] = jnp.where(m, _xperm(a, s, lane), b)
        vecs = nv
    return vecs


def _blocks_loop(sta_buf, outb_buf, nblk, lane, size):
    def blocks(g, carry):
        vecs = [sta_buf[pl.ds(k * size + g * L, L)] for k in range(D)]
        vecs = _transpose16(vecs, lane)
        for i in range(L):
            outb_buf[2 * g + (i // 8), pl.ds((i % 8) * D, D)] = vecs[i]
        return carry

    lax.fori_loop(0, nblk, blocks, 0)


def _phase_a(src, dst, nfull, t_count, w, stas, outbs, sem_st, sem_wb, lane):
    def fire_stage(c, buf):
        u0 = c * S
        for k in range(D):
            pltpu.async_copy(src.at[k, pl.ds(u0, S)],
                             stas[buf].at[pl.ds(k * S, S)],
                             sem_st.at[buf])

    def wait_stage(buf):
        pltpu.make_async_copy(src.at[0, pl.ds(0, D * S)],
                              stas[buf], sem_st.at[buf]).wait()

    def wait_wb(buf):
        pltpu.make_async_copy(outbs[0],
                              dst.at[pl.ds(0, SR), :], sem_wb.at[buf]).wait()

    c0 = w

    @pl.when(c0 < nfull)
    def _():
        fire_stage(c0, 0)

    def process(t, buf):
        c = w + NW * t

        @pl.when(c < nfull)
        def _():
            wait_stage(buf)
            nxt = c + NW

            @pl.when(nxt < nfull)
            def _():
                fire_stage(nxt, 1 - buf)

            @pl.when(t >= 2)
            def _():
                wait_wb(buf)

            _blocks_loop(stas[buf], outbs[buf], S // L, lane, S)
            pltpu.async_copy(outbs[buf],
                             dst.at[pl.ds(c * SR, SR), :],
                             sem_wb.at[buf])

    def body(th, carry):
        process(2 * th, 0)
        process(2 * th + 1, 1)
        return carry

    lax.fori_loop(0, (t_count + 1) // 2, body, 0)

    nt = jnp.where(w < nfull, (nfull - 1 - w) // NW + 1, 0)

    @pl.when(nt >= 2)
    def _():
        wait_wb(nt & 1)

    @pl.when(nt >= 1)
    def _():
        wait_wb((nt + 1) & 1)


@functools.partial(
    pl.kernel,
    out_type=(jax.ShapeDtypeStruct((NU // 8, 128), jnp.float32),
              jax.ShapeDtypeStruct((NSK // 8, 128), jnp.float32)),
    mesh=plsc.VectorSubcoreMesh(core_axis_name="c", subcore_axis_name="s"),
    scratch_types=[
        pltpu.VMEM((D * S,), jnp.float32),
        pltpu.VMEM((D * S,), jnp.float32),
        pltpu.VMEM((SR, 128), jnp.float32),
        pltpu.VMEM((SR, 128), jnp.float32),
        pltpu.SemaphoreType.DMA((2,)),
        pltpu.SemaphoreType.DMA((2,)),
    ],
)
def _transpose_kernel(ut_t, st_t, u_tail, s_tail, ut8, st8,
                      sta0, sta1, outb0, outb1, sem_st, sem_wb):
    w = lax.axis_index("s") * NC + lax.axis_index("c")
    lane = lax.iota(jnp.int32, L)
    stas, outbs = (sta0, sta1), (outb0, outb1)
    _phase_a(ut_t, ut8, NFULL_U, TU, w, stas, outbs, sem_st, sem_wb, lane)
    _phase_a(st_t, st8, NFULL_S, TS, w, stas, outbs, sem_st, sem_wb, lane)

    @pl.when(w == 0)
    def _():
        pltpu.sync_copy(u_tail, ut8.at[pl.ds(NFULL_U * SR, 8), :])

    @pl.when(w == 1)
    def _():
        pltpu.sync_copy(s_tail, st8.at[pl.ds(NFULL_S * SR, 20), :])


def _sc_body(uhi_hbm, ulo_hbm, shi_hbm, slo_hbm, ut_hbm, st_hbm, wb_hbm,
             out_hbm, hi_u, hi_s, lo_u, lo_s, ru, rs, out_v, wv, sems):
    wid = lax.axis_index("s") * NC + lax.axis_index("c")
    base = wid * BPW

    pltpu.sync_copy(wb_hbm, wv)
    pltpu.sync_copy(uhi_hbm.at[wid], hi_u)
    pltpu.sync_copy(shi_hbm.at[wid], hi_s)
    pltpu.sync_copy(ulo_hbm.at[wid], lo_u)
    pltpu.sync_copy(slo_hbm.at[wid], lo_s)

    lane = lax.iota(jnp.int32, L)
    w_u = wv[0]
    w_s = wv[1]
    bb = wv[2]

    def xperm(x, s):
        return jnp.take_along_axis(x, lane ^ s, axis=0,
                                   mode="promise_in_bounds")

    def combine(x, y, s):
        return jnp.where((lane & s) == 0, x + xperm(x, s), y + xperm(y, s))

    def fire(c):
        buf = c % 2
        cu = pltpu.async_copy(ut_hbm.at[hi_u.at[c]], ru.at[buf], sems.at[buf])
        cs = pltpu.async_copy(st_hbm.at[hi_s.at[c]], rs.at[buf], sems.at[buf])
        return cu, cs

    pending = fire(0)
    for c in range(NCHUNK):
        for cp in pending:
            cp.wait()
        if c + 1 < NCHUNK:
            pending = fire(c + 1)
        buf = c % 2

        def group(g, carry):
            lov_u = lo_u[c * GPC + g]
            lov_s = lo_s[c * GPC + g]
            ps = []
            for j in range(L):
                ur = ru[buf, g * L + j, pl.ds(lov_u[j], L)]
                sr = rs[buf, g * L + j, pl.ds(lov_s[j], L)]
                ps.append(ur * w_u + sr * w_s)
            ps = [combine(ps[i], ps[i + 8], 8) for i in range(8)]
            ps = [combine(ps[i], ps[i + 4], 4) for i in range(4)]
            ps = [combine(ps[i], ps[i + 2], 2) for i in range(2)]
            acc = combine(ps[0], ps[1], 1) + bb
            out_v[pl.ds((c * GPC + g) * L, L)] = acc
            return carry

        lax.fori_loop(0, GPC, group, 0)

    pltpu.sync_copy(out_v, out_hbm.at[pl.ds(base, BPW)])


@functools.partial(
    pl.kernel,
    out_type=jax.ShapeDtypeStruct((B,), jnp.float32),
    mesh=plsc.VectorSubcoreMesh(core_axis_name="c", subcore_axis_name="s"),
    scratch_types=[
        pltpu.VMEM((NCHUNK, CHUNK), jnp.int32),
        pltpu.VMEM((NCHUNK, CHUNK), jnp.int32),
        pltpu.VMEM((BPW // L, L), jnp.int32),
        pltpu.VMEM((BPW // L, L), jnp.int32),
        pltpu.VMEM((2, CHUNK, 128), jnp.float32),
        pltpu.VMEM((2, CHUNK, 128), jnp.float32),
        pltpu.VMEM((BPW,), jnp.float32),
        pltpu.VMEM((3, L), jnp.float32),
        pltpu.SemaphoreType.DMA((2,)),
    ],
)
def _sc_kernel(uhi_hbm, ulo_hbm, shi_hbm, slo_hbm, ut_hbm, st_hbm, wb_hbm,
               out_hbm, hi_u, hi_s, lo_u, lo_s, ru, rs, out_v, wv, sems):
    _sc_body(uhi_hbm, ulo_hbm, shi_hbm, slo_hbm, ut_hbm, st_hbm, wb_hbm,
             out_hbm, hi_u, hi_s, lo_u, lo_s, ru, rs, out_v, wv, sems)


def kernel(user, skill, user_table, skill_table, W, b):
    user = user.astype(jnp.int32)
    skill = skill.astype(jnp.int32)
    uhi = (user >> 3).reshape(NW, NCHUNK, CHUNK)
    shi = (skill >> 3).reshape(NW, NCHUNK, CHUNK)
    ulo = ((user & 7) * D).reshape(NW, BPW // L, L)
    slo = ((skill & 7) * D).reshape(NW, BPW // L, L)
    wb = jnp.stack(
        [W[:D, 0], W[D:, 0], jnp.broadcast_to(b.astype(jnp.float32), (L,))]
    ).astype(jnp.float32)
    u_tail = user_table[NFULL_U * S:].reshape(8, 128)
    s_tail = skill_table[NFULL_S * S:].reshape(20, 128)
    ut8, st8 = _transpose_kernel(user_table.T, skill_table.T, u_tail, s_tail)
    return _sc_kernel(uhi, ulo, shi, slo, ut8, st8, wb)


# speedup vs baseline: 1.2078x; 1.2078x over previous
import functools

import jax
import jax.numpy as jnp
from jax import lax
from jax.experimental import pallas as pl
from jax.experimental.pallas import tpu as pltpu
from jax.experimental.pallas import tpu_sc as plsc

B = 16384
D = 16
L = 16
NC = 2
NS = 16
NW = NC * NS
BPW = B // NW
NCHUNK = 4
CHUNK = BPW // NCHUNK

NU = 1000000
NSK = 100000
S = 512
NFULL_U = NU // S
NFULL_S = NSK // S
TU = (NFULL_U + NW - 1) // NW
TS = (NFULL_S + NW - 1) // NW


def _xperm(x, s, lane):
    return jnp.take_along_axis(x, lane ^ s, axis=0, mode="promise_in_bounds")


def _sum16(ps, lane):
    def combine(x, y, s):
        return jnp.where((lane & s) == 0, x + _xperm(x, s, lane),
                         y + _xperm(y, s, lane))

    ps = [combine(ps[i], ps[i + 8], 8) for i in range(8)]
    ps = [combine(ps[i], ps[i + 4], 4) for i in range(4)]
    ps = [combine(ps[i], ps[i + 2], 2) for i in range(2)]
    return combine(ps[0], ps[1], 1)


def _phase_a(src, dst, w_rows, nfull, t_count, w, stas, ods, sem_st, sem_wb):
    def fire_stage(c, buf):
        u0 = c * S
        for k in range(D):
            pltpu.async_copy(src.at[k, pl.ds(u0, S)],
                             stas[buf].at[pl.ds(k * S, S)],
                             sem_st.at[buf])

    def wait_stage(buf):
        pltpu.make_async_copy(src.at[0, pl.ds(0, D * S)],
                              stas[buf], sem_st.at[buf]).wait()

    def wait_wb(buf):
        pltpu.make_async_copy(ods[0], dst.at[pl.ds(0, S)],
                              sem_wb.at[buf]).wait()

    @pl.when(w < nfull)
    def _():
        fire_stage(w, 0)

    def process(t, buf):
        c = w + NW * t

        @pl.when(c < nfull)
        def _():
            wait_stage(buf)
            nxt = c + NW

            @pl.when(nxt < nfull)
            def _():
                fire_stage(nxt, 1 - buf)

            @pl.when(t >= 2)
            def _():
                wait_wb(buf)

            def groups(g, carry):
                acc = stas[buf][pl.ds(g * L, L)] * w_rows[0]
                for k in range(1, D):
                    acc = acc + stas[buf][pl.ds(k * S + g * L, L)] * w_rows[k]
                ods[buf][pl.ds(g * L, L)] = acc
                return carry

            lax.fori_loop(0, S // L, groups, 0)
            pltpu.async_copy(ods[buf], dst.at[pl.ds(c * S, S)],
                             sem_wb.at[buf])
        return None

    def body(th, carry):
        process(2 * th, 0)
        process(2 * th + 1, 1)
        return carry

    lax.fori_loop(0, (t_count + 1) // 2, body, 0)

    nt = jnp.where(w < nfull, (nfull - 1 - w) // NW + 1, 0)

    @pl.when(nt >= 2)
    def _():
        wait_wb(nt & 1)

    @pl.when(nt >= 1)
    def _():
        wait_wb((nt + 1) & 1)


def _tail_dots(tail, dst, row0, n, w_vec, sta, od, lane):
    pltpu.sync_copy(tail, sta.at[pl.ds(0, n * D)])
    for g in range(n // L):
        ps = [sta[pl.ds((g * L + j) * D, D)] * w_vec for j in range(L)]
        od[pl.ds(g * L, L)] = _sum16(ps, lane)
    pltpu.sync_copy(od.at[pl.ds(0, n)], dst.at[pl.ds(row0, n)])


@functools.partial(
    pl.kernel,
    out_type=(jax.ShapeDtypeStruct((NU,), jnp.float32),
              jax.ShapeDtypeStruct((NSK,), jnp.float32)),
    mesh=plsc.VectorSubcoreMesh(core_axis_name="c", subcore_axis_name="s"),
    scratch_types=[
        pltpu.VMEM((D * S,), jnp.float32),
        pltpu.VMEM((D * S,), jnp.float32),
        pltpu.VMEM((S,), jnp.float32),
        pltpu.VMEM((S,), jnp.float32),
        pltpu.VMEM((2 * D + 3, L), jnp.float32),
        pltpu.SemaphoreType.DMA((2,)),
        pltpu.SemaphoreType.DMA((2,)),
    ],
)
def _proj_kernel(ut_t, st_t, u_tail, s_tail, wb_hbm, udot, sdot,
                 sta0, sta1, od0, od1, wv, sem_st, sem_wb):
    w = lax.axis_index("s") * NC + lax.axis_index("c")
    lane = lax.iota(jnp.int32, L)
    pltpu.sync_copy(wb_hbm, wv)
    wu_rows = [wv[k] for k in range(D)]
    ws_rows = [wv[D + k] for k in range(D)]
    stas, ods = (sta0, sta1), (od0, od1)
    _phase_a(ut_t, udot, wu_rows, NFULL_U, TU, w, stas, ods, sem_st, sem_wb)
    _phase_a(st_t, sdot, ws_rows, NFULL_S, TS, w, stas, ods, sem_st, sem_wb)

    @pl.when(w == 0)
    def _():
        _tail_dots(u_tail, udot, NFULL_U * S, 64, wv[2 * D + 1], sta0, od0,
                   lane)

    @pl.when(w == 1)
    def _():
        _tail_dots(s_tail, sdot, NFULL_S * S, 160, wv[2 * D + 2], sta0, od0,
                   lane)


@functools.partial(
    pl.kernel,
    out_type=jax.ShapeDtypeStruct((B,), jnp.float32),
    mesh=plsc.VectorSubcoreMesh(core_axis_name="c", subcore_axis_name="s"),
    scratch_types=[
        pltpu.VMEM((NCHUNK, CHUNK), jnp.int32),
        pltpu.VMEM((NCHUNK, CHUNK), jnp.int32),
        pltpu.VMEM((BPW,), jnp.float32),
        pltpu.VMEM((BPW,), jnp.float32),
        pltpu.VMEM((BPW,), jnp.float32),
        pltpu.VMEM((2 * D + 3, L), jnp.float32),
        pltpu.SemaphoreType.DMA,
    ],
)
def _gather_kernel(user_hbm, skill_hbm, udot_hbm, sdot_hbm, wb_hbm, out_hbm,
                   ui, si, gu, gs, out_v, wv, sem):
    wid = lax.axis_index("s") * NC + lax.axis_index("c")
    base = wid * BPW

    pltpu.sync_copy(wb_hbm, wv)
    pltpu.sync_copy(user_hbm.at[wid], ui)
    pltpu.sync_copy(skill_hbm.at[wid], si)

    copies = []
    for j in range(NCHUNK):
        copies.append(pltpu.async_copy(
            udot_hbm.at[ui.at[j]], gu.at[pl.ds(j * CHUNK, CHUNK)], sem))
        copies.append(pltpu.async_copy(
            sdot_hbm.at[si.at[j]], gs.at[pl.ds(j * CHUNK, CHUNK)], sem))
    for c in copies:
        c.wait()

    bb = wv[2 * D]

    def groups(g, carry):
        out_v[pl.ds(g * L, L)] = (gu[pl.ds(g * L, L)] + gs[pl.ds(g * L, L)]
                                  + bb)
        return carry

    lax.fori_loop(0, BPW // L, groups, 0)
    pltpu.sync_copy(out_v, out_hbm.at[pl.ds(base, BPW)])


def kernel(user, skill, user_table, skill_table, W, b):
    user_r = user.astype(jnp.int32).reshape(NW, NCHUNK, CHUNK)
    skill_r = skill.astype(jnp.int32).reshape(NW, NCHUNK, CHUNK)
    u_tail = user_table[NFULL_U * S:].reshape(64 * D)
    s_tail = skill_table[NFULL_S * S:].reshape(160 * D)
    wb = jnp.concatenate([
        jnp.broadcast_to(
            jnp.concatenate([W[:, 0], b.astype(jnp.float32)])[:, None],
            (2 * D + 1, L),
        ),
        W[:D, 0][None, :],
        W[D:, 0][None, :],
    ]).astype(jnp.float32)
    udot, sdot = _proj_kernel(user_table.T, skill_table.T, u_tail, s_tail, wb)
    return _gather_kernel(user_r, skill_r, udot, sdot, wb)


# speedup vs baseline: 1.2108x; 1.0024x over previous
import functools

import jax
import jax.numpy as jnp
from jax import lax
from jax.experimental import pallas as pl
from jax.experimental.pallas import tpu as pltpu
from jax.experimental.pallas import tpu_sc as plsc

B = 16384
D = 16
L = 16
NC = 2
NS = 16
NW = NC * NS
BPW = B // NW
NCHUNK = 4
CHUNK = BPW // NCHUNK

NU = 1000000
NSK = 100000
S = 512
NFULL_U = NU // S
NFULL_S = NSK // S
TU = (NFULL_U + NW - 1) // NW
TS = (NFULL_S + NW - 1) // NW


def _xperm(x, s, lane):
    return jnp.take_along_axis(x, lane ^ s, axis=0, mode="promise_in_bounds")


def _sum16(ps, lane):
    def combine(x, y, s):
        return jnp.where((lane & s) == 0, x + _xperm(x, s, lane),
                         y + _xperm(y, s, lane))

    ps = [combine(ps[i], ps[i + 8], 8) for i in range(8)]
    ps = [combine(ps[i], ps[i + 4], 4) for i in range(4)]
    ps = [combine(ps[i], ps[i + 2], 2) for i in range(2)]
    return combine(ps[0], ps[1], 1)


def _phase_a(src, dst, w_rows, nfull, t_count, w, stas, ods, sem_st, sem_wb):
    def fire_stage(c, buf):
        pltpu.async_copy(src.at[:, pl.ds(c * S, S)], stas[buf],
                         sem_st.at[buf])

    def wait_stage(buf):
        pltpu.make_async_copy(src.at[:, pl.ds(0, S)],
                              stas[buf], sem_st.at[buf]).wait()

    def wait_wb(buf):
        pltpu.make_async_copy(ods[0], dst.at[pl.ds(0, S)],
                              sem_wb.at[buf]).wait()

    @pl.when(w < nfull)
    def _():
        fire_stage(w, 0)

    def process(t, buf):
        c = w + NW * t

        @pl.when(c < nfull)
        def _():
            wait_stage(buf)
            nxt = c + NW

            @pl.when(nxt < nfull)
            def _():
                fire_stage(nxt, 1 - buf)

            @pl.when(t >= 2)
            def _():
                wait_wb(buf)

            def groups(g, carry):
                acc = stas[buf][0, pl.ds(g * L, L)] * w_rows[0]
                for k in range(1, D):
                    acc = acc + stas[buf][k, pl.ds(g * L, L)] * w_rows[k]
                ods[buf][pl.ds(g * L, L)] = acc
                return carry

            lax.fori_loop(0, S // L, groups, 0)
            pltpu.async_copy(ods[buf], dst.at[pl.ds(c * S, S)],
                             sem_wb.at[buf])
        return None

    def body(th, carry):
        process(2 * th, 0)
        process(2 * th + 1, 1)
        return carry

    lax.fori_loop(0, (t_count + 1) // 2, body, 0)

    nt = jnp.where(w < nfull, (nfull - 1 - w) // NW + 1, 0)

    @pl.when(nt >= 2)
    def _():
        wait_wb(nt & 1)

    @pl.when(nt >= 1)
    def _():
        wait_wb((nt + 1) & 1)


def _tail_dots(tail, dst, row0, n, w_vec, tb, od, lane):
    pltpu.sync_copy(tail, tb.at[pl.ds(0, n * D)])
    for g in range(n // L):
        ps = [tb[pl.ds((g * L + j) * D, D)] * w_vec for j in range(L)]
        od[pl.ds(g * L, L)] = _sum16(ps, lane)
    pltpu.sync_copy(od.at[pl.ds(0, n)], dst.at[pl.ds(row0, n)])


@functools.partial(
    pl.kernel,
    out_type=(jax.ShapeDtypeStruct((NU,), jnp.float32),
              jax.ShapeDtypeStruct((NSK,), jnp.float32)),
    mesh=plsc.VectorSubcoreMesh(core_axis_name="c", subcore_axis_name="s"),
    scratch_types=[
        pltpu.VMEM((D, S), jnp.float32),
        pltpu.VMEM((D, S), jnp.float32),
        pltpu.VMEM((S,), jnp.float32),
        pltpu.VMEM((S,), jnp.float32),
        pltpu.VMEM((160 * D,), jnp.float32),
        pltpu.VMEM((2 * D + 3, L), jnp.float32),
        pltpu.SemaphoreType.DMA((2,)),
        pltpu.SemaphoreType.DMA((2,)),
    ],
)
def _proj_kernel(ut_t, st_t, u_tail, s_tail, wb_hbm, udot, sdot,
                 sta0, sta1, od0, od1, tb, wv, sem_st, sem_wb):
    w = lax.axis_index("s") * NC + lax.axis_index("c")
    lane = lax.iota(jnp.int32, L)
    pltpu.sync_copy(wb_hbm, wv)
    wu_rows = [wv[k] for k in range(D)]
    ws_rows = [wv[D + k] for k in range(D)]
    stas, ods = (sta0, sta1), (od0, od1)
    _phase_a(ut_t, udot, wu_rows, NFULL_U, TU, w, stas, ods, sem_st, sem_wb)
    _phase_a(st_t, sdot, ws_rows, NFULL_S, TS, w, stas, ods, sem_st, sem_wb)

    @pl.when(w == 0)
    def _():
        _tail_dots(u_tail, udot, NFULL_U * S, 64, wv[2 * D + 1], tb, od0,
                   lane)

    @pl.when(w == 1)
    def _():
        _tail_dots(s_tail, sdot, NFULL_S * S, 160, wv[2 * D + 2], tb, od0,
                   lane)


@functools.partial(
    pl.kernel,
    out_type=jax.ShapeDtypeStruct((B,), jnp.float32),
    mesh=plsc.VectorSubcoreMesh(core_axis_name="c", subcore_axis_name="s"),
    scratch_types=[
        pltpu.VMEM((NCHUNK, CHUNK), jnp.int32),
        pltpu.VMEM((NCHUNK, CHUNK), jnp.int32),
        pltpu.VMEM((BPW,), jnp.float32),
        pltpu.VMEM((BPW,), jnp.float32),
        pltpu.VMEM((BPW,), jnp.float32),
        pltpu.VMEM((2 * D + 3, L), jnp.float32),
        pltpu.SemaphoreType.DMA,
    ],
)
def _gather_kernel(user_hbm, skill_hbm, udot_hbm, sdot_hbm, wb_hbm, out_hbm,
                   ui, si, gu, gs, out_v, wv, sem):
    wid = lax.axis_index("s") * NC + lax.axis_index("c")
    base = wid * BPW

    pltpu.sync_copy(wb_hbm, wv)
    pltpu.sync_copy(user_hbm.at[wid], ui)
    pltpu.sync_copy(skill_hbm.at[wid], si)

    copies = []
    for j in range(NCHUNK):
        copies.append(pltpu.async_copy(
            udot_hbm.at[ui.at[j]], gu.at[pl.ds(j * CHUNK, CHUNK)], sem))
        copies.append(pltpu.async_copy(
            sdot_hbm.at[si.at[j]], gs.at[pl.ds(j * CHUNK, CHUNK)], sem))
    for c in copies:
        c.wait()

    bb = wv[2 * D]

    def groups(g, carry):
        out_v[pl.ds(g * L, L)] = (gu[pl.ds(g * L, L)] + gs[pl.ds(g * L, L)]
                                  + bb)
        return carry

    lax.fori_loop(0, BPW // L, groups, 0)
    pltpu.sync_copy(out_v, out_hbm.at[pl.ds(base, BPW)])


def kernel(user, skill, user_table, skill_table, W, b):
    user_r = user.astype(jnp.int32).reshape(NW, NCHUNK, CHUNK)
    skill_r = skill.astype(jnp.int32).reshape(NW, NCHUNK, CHUNK)
    u_tail = user_table[NFULL_U * S:].reshape(64 * D)
    s_tail = skill_table[NFULL_S * S:].reshape(160 * D)
    wb = jnp.concatenate([
        jnp.broadcast_to(
            jnp.concatenate([W[:, 0], b.astype(jnp.float32)])[:, None],
            (2 * D + 1, L),
        ),
        W[:D, 0][None, :],
        W[D:, 0][None, :],
    ]).astype(jnp.float32)
    udot, sdot = _proj_kernel(user_table.T, skill_table.T, u_tail, s_tail, wb)
    return _gather_kernel(user_r, skill_r, udot, sdot, wb)


# speedup vs baseline: 1.2215x; 1.0089x over previous
import functools

import jax
import jax.numpy as jnp
from jax import lax
from jax.experimental import pallas as pl
from jax.experimental.pallas import tpu as pltpu
from jax.experimental.pallas import tpu_sc as plsc

B = 16384
D = 16
L = 16
NC = 2
NS = 16
NW = NC * NS
BPW = B // NW
NCHUNK = 4
CHUNK = BPW // NCHUNK

NU = 1000000
NSK = 100000
S = 512
NFULL_U = NU // S
NFULL_S = NSK // S
TU = (NFULL_U + NW - 1) // NW
TS = (NFULL_S + NW - 1) // NW


def _xperm(x, s, lane):
    return jnp.take_along_axis(x, lane ^ s, axis=0, mode="promise_in_bounds")


def _sum16(ps, lane):
    def combine(x, y, s):
        return jnp.where((lane & s) == 0, x + _xperm(x, s, lane),
                         y + _xperm(y, s, lane))

    ps = [combine(ps[i], ps[i + 8], 8) for i in range(8)]
    ps = [combine(ps[i], ps[i + 4], 4) for i in range(4)]
    ps = [combine(ps[i], ps[i + 2], 2) for i in range(2)]
    return combine(ps[0], ps[1], 1)


def _phase_a(src, dst, w_rows, nfull, t_count, w, stas, ods, sem_st, sem_wb):
    def fire_stage(c, buf):
        pltpu.async_copy(src.at[:, pl.ds(c * S, S)], stas[buf],
                         sem_st.at[buf])

    def wait_stage(buf):
        pltpu.make_async_copy(src.at[:, pl.ds(0, S)],
                              stas[buf], sem_st.at[buf]).wait()

    def wait_wb(buf):
        pltpu.make_async_copy(ods[0], dst.at[pl.ds(0, S)],
                              sem_wb.at[buf]).wait()

    @pl.when(w < nfull)
    def _():
        fire_stage(w, 0)

    def process(t, buf):
        c = w + NW * t

        @pl.when(c < nfull)
        def _():
            wait_stage(buf)
            nxt = c + NW

            @pl.when(nxt < nfull)
            def _():
                fire_stage(nxt, 1 - buf)

            @pl.when(t >= 2)
            def _():
                wait_wb(buf)

            @plsc.parallel_loop(0, S // L, unroll=2)
            def groups(g):
                vs = [stas[buf][k, pl.ds(g * L, L)] * w_rows[k]
                      for k in range(D)]
                while len(vs) > 1:
                    vs = [vs[i] + vs[i + 1] for i in range(0, len(vs), 2)]
                ods[buf][pl.ds(g * L, L)] = vs[0]
            pltpu.async_copy(ods[buf], dst.at[pl.ds(c * S, S)],
                             sem_wb.at[buf])
        return None

    def body(th, carry):
        process(2 * th, 0)
        process(2 * th + 1, 1)
        return carry

    lax.fori_loop(0, (t_count + 1) // 2, body, 0)

    nt = jnp.where(w < nfull, (nfull - 1 - w) // NW + 1, 0)

    @pl.when(nt >= 2)
    def _():
        wait_wb(nt & 1)

    @pl.when(nt >= 1)
    def _():
        wait_wb((nt + 1) & 1)


def _tail_dots(tail, dst, row0, n, w_vec, tb, od, lane):
    pltpu.sync_copy(tail, tb.at[pl.ds(0, n * D)])
    for g in range(n // L):
        ps = [tb[pl.ds((g * L + j) * D, D)] * w_vec for j in range(L)]
        od[pl.ds(g * L, L)] = _sum16(ps, lane)
    pltpu.sync_copy(od.at[pl.ds(0, n)], dst.at[pl.ds(row0, n)])


@functools.partial(
    pl.kernel,
    out_type=(jax.ShapeDtypeStruct((NU,), jnp.float32),
              jax.ShapeDtypeStruct((NSK,), jnp.float32)),
    mesh=plsc.VectorSubcoreMesh(core_axis_name="c", subcore_axis_name="s"),
    scratch_types=[
        pltpu.VMEM((D, S), jnp.float32),
        pltpu.VMEM((D, S), jnp.float32),
        pltpu.VMEM((S,), jnp.float32),
        pltpu.VMEM((S,), jnp.float32),
        pltpu.VMEM((160 * D,), jnp.float32),
        pltpu.VMEM((2 * D + 3, L), jnp.float32),
        pltpu.SemaphoreType.DMA((2,)),
        pltpu.SemaphoreType.DMA((2,)),
    ],
)
def _proj_kernel(ut_t, st_t, u_tail, s_tail, wb_hbm, udot, sdot,
                 sta0, sta1, od0, od1, tb, wv, sem_st, sem_wb):
    w = lax.axis_index("s") * NC + lax.axis_index("c")
    lane = lax.iota(jnp.int32, L)
    pltpu.sync_copy(wb_hbm, wv)
    wu_rows = [wv[k] for k in range(D)]
    ws_rows = [wv[D + k] for k in range(D)]
    stas, ods = (sta0, sta1), (od0, od1)
    _phase_a(ut_t, udot, wu_rows, NFULL_U, TU, w, stas, ods, sem_st, sem_wb)
    _phase_a(st_t, sdot, ws_rows, NFULL_S, TS, w, stas, ods, sem_st, sem_wb)

    @pl.when(w == 0)
    def _():
        _tail_dots(u_tail, udot, NFULL_U * S, 64, wv[2 * D + 1], tb, od0,
                   lane)

    @pl.when(w == 1)
    def _():
        _tail_dots(s_tail, sdot, NFULL_S * S, 160, wv[2 * D + 2], tb, od0,
                   lane)


@functools.partial(
    pl.kernel,
    out_type=jax.ShapeDtypeStruct((B,), jnp.float32),
    mesh=plsc.VectorSubcoreMesh(core_axis_name="c", subcore_axis_name="s"),
    scratch_types=[
        pltpu.VMEM((NCHUNK, CHUNK), jnp.int32),
        pltpu.VMEM((NCHUNK, CHUNK), jnp.int32),
        pltpu.VMEM((BPW,), jnp.float32),
        pltpu.VMEM((BPW,), jnp.float32),
        pltpu.VMEM((BPW,), jnp.float32),
        pltpu.VMEM((2 * D + 3, L), jnp.float32),
        pltpu.SemaphoreType.DMA,
    ],
)
def _gather_kernel(user_hbm, skill_hbm, udot_hbm, sdot_hbm, wb_hbm, out_hbm,
                   ui, si, gu, gs, out_v, wv, sem):
    wid = lax.axis_index("s") * NC + lax.axis_index("c")
    base = wid * BPW

    pltpu.sync_copy(wb_hbm, wv)
    pltpu.sync_copy(user_hbm.at[wid], ui)
    pltpu.sync_copy(skill_hbm.at[wid], si)

    copies = []
    for j in range(NCHUNK):
        copies.append(pltpu.async_copy(
            udot_hbm.at[ui.at[j]], gu.at[pl.ds(j * CHUNK, CHUNK)], sem))
        copies.append(pltpu.async_copy(
            sdot_hbm.at[si.at[j]], gs.at[pl.ds(j * CHUNK, CHUNK)], sem))
    for c in copies:
        c.wait()

    bb = wv[2 * D]

    def groups(g, carry):
        out_v[pl.ds(g * L, L)] = (gu[pl.ds(g * L, L)] + gs[pl.ds(g * L, L)]
                                  + bb)
        return carry

    lax.fori_loop(0, BPW // L, groups, 0)
    pltpu.sync_copy(out_v, out_hbm.at[pl.ds(base, BPW)])


def kernel(user, skill, user_table, skill_table, W, b):
    user_r = user.astype(jnp.int32).reshape(NW, NCHUNK, CHUNK)
    skill_r = skill.astype(jnp.int32).reshape(NW, NCHUNK, CHUNK)
    u_tail = user_table[NFULL_U * S:].reshape(64 * D)
    s_tail = skill_table[NFULL_S * S:].reshape(160 * D)
    wb = jnp.concatenate([
        jnp.broadcast_to(
            jnp.concatenate([W[:, 0], b.astype(jnp.float32)])[:, None],
            (2 * D + 1, L),
        ),
        W[:D, 0][None, :],
        W[D:, 0][None, :],
    ]).astype(jnp.float32)
    udot, sdot = _proj_kernel(user_table.T, skill_table.T, u_tail, s_tail, wb)
    return _gather_kernel(user_r, skill_r, udot, sdot, wb)


# speedup vs baseline: 1.5066x; 1.2334x over previous
import functools

import jax
import jax.numpy as jnp
from jax import lax
from jax.experimental import pallas as pl
from jax.experimental.pallas import tpu as pltpu
from jax.experimental.pallas import tpu_sc as plsc

B = 16384
D = 16
L = 16
NC = 2
NS = 16
NW = NC * NS
BPW = B // NW
NCHUNK = 4
CHUNK = BPW // NCHUNK

NU = 1000000
NSK = 100000
S = 1024
NFULL_U = NU // S
NFULL_S = NSK // S
TU = (NFULL_U + NW - 1) // NW
TS = (NFULL_S + NW - 1) // NW


def _xperm(x, s, lane):
    return jnp.take_along_axis(x, lane ^ s, axis=0, mode="promise_in_bounds")


def _sum16(ps, lane):
    def combine(x, y, s):
        return jnp.where((lane & s) == 0, x + _xperm(x, s, lane),
                         y + _xperm(y, s, lane))

    ps = [combine(ps[i], ps[i + 8], 8) for i in range(8)]
    ps = [combine(ps[i], ps[i + 4], 4) for i in range(4)]
    ps = [combine(ps[i], ps[i + 2], 2) for i in range(2)]
    return combine(ps[0], ps[1], 1)


def _phase_a(src, dst, w_rows, nfull, t_count, w, stas, ods, sem_st, sem_wb):
    def fire_stage(c, buf):
        pltpu.async_copy(src.at[:, pl.ds(c * S, S)], stas[buf],
                         sem_st.at[buf])

    def wait_stage(buf):
        pltpu.make_async_copy(src.at[:, pl.ds(0, S)],
                              stas[buf], sem_st.at[buf]).wait()

    def wait_wb(buf):
        pltpu.make_async_copy(ods[0], dst.at[pl.ds(0, S)],
                              sem_wb.at[buf]).wait()

    @pl.when(w < nfull)
    def _():
        fire_stage(w, 0)

    def process(t, buf):
        c = w + NW * t

        @pl.when(c < nfull)
        def _():
            wait_stage(buf)
            nxt = c + NW

            @pl.when(nxt < nfull)
            def _():
                fire_stage(nxt, 1 - buf)

            @pl.when(t >= 2)
            def _():
                wait_wb(buf)

            @plsc.parallel_loop(0, S // L, unroll=2)
            def groups(g):
                vs = [stas[buf][k, pl.ds(g * L, L)] * w_rows[k]
                      for k in range(D)]
                while len(vs) > 1:
                    vs = [vs[i] + vs[i + 1] for i in range(0, len(vs), 2)]
                ods[buf][pl.ds(g * L, L)] = vs[0]
            pltpu.async_copy(ods[buf], dst.at[pl.ds(c * S, S)],
                             sem_wb.at[buf])
        return None

    def body(th, carry):
        process(2 * th, 0)
        process(2 * th + 1, 1)
        return carry

    lax.fori_loop(0, (t_count + 1) // 2, body, 0)

    nt = jnp.where(w < nfull, (nfull - 1 - w) // NW + 1, 0)

    @pl.when(nt >= 2)
    def _():
        wait_wb(nt & 1)

    @pl.when(nt >= 1)
    def _():
        wait_wb((nt + 1) & 1)


def _tail_dots(tail, dst, row0, n, w_vec, tb, od, lane):
    pltpu.sync_copy(tail, tb.at[pl.ds(0, n * D)])

    def group(g, carry):
        ps = [tb[pl.ds((g * L + j) * D, D)] * w_vec for j in range(L)]
        od[pl.ds(g * L, L)] = _sum16(ps, lane)
        return carry

    lax.fori_loop(0, n // L, group, 0)
    pltpu.sync_copy(od.at[pl.ds(0, n)], dst.at[pl.ds(row0, n)])


@functools.partial(
    pl.kernel,
    out_type=(jax.ShapeDtypeStruct((NU,), jnp.float32),
              jax.ShapeDtypeStruct((NSK,), jnp.float32)),
    mesh=plsc.VectorSubcoreMesh(core_axis_name="c", subcore_axis_name="s"),
    scratch_types=[
        pltpu.VMEM((D, S), jnp.float32),
        pltpu.VMEM((D, S), jnp.float32),
        pltpu.VMEM((S,), jnp.float32),
        pltpu.VMEM((S,), jnp.float32),
        pltpu.VMEM(((NSK - NFULL_S * S) * D,), jnp.float32),
        pltpu.VMEM((2 * D + 3, L), jnp.float32),
        pltpu.SemaphoreType.DMA((2,)),
        pltpu.SemaphoreType.DMA((2,)),
    ],
)
def _proj_kernel(ut_t, st_t, u_tail, s_tail, wb_hbm, udot, sdot,
                 sta0, sta1, od0, od1, tb, wv, sem_st, sem_wb):
    w = lax.axis_index("s") * NC + lax.axis_index("c")
    lane = lax.iota(jnp.int32, L)
    pltpu.sync_copy(wb_hbm, wv)
    wu_rows = [wv[k] for k in range(D)]
    ws_rows = [wv[D + k] for k in range(D)]
    stas, ods = (sta0, sta1), (od0, od1)
    _phase_a(ut_t, udot, wu_rows, NFULL_U, TU, w, stas, ods, sem_st, sem_wb)
    _phase_a(st_t, sdot, ws_rows, NFULL_S, TS, w, stas, ods, sem_st, sem_wb)

    @pl.when(w == 0)
    def _():
        _tail_dots(u_tail, udot, NFULL_U * S, NU - NFULL_U * S,
                   wv[2 * D + 1], tb, od0, lane)

    @pl.when(w == 1)
    def _():
        _tail_dots(s_tail, sdot, NFULL_S * S, NSK - NFULL_S * S,
                   wv[2 * D + 2], tb, od0, lane)


@functools.partial(
    pl.kernel,
    out_type=jax.ShapeDtypeStruct((B,), jnp.float32),
    mesh=plsc.VectorSubcoreMesh(core_axis_name="c", subcore_axis_name="s"),
    scratch_types=[
        pltpu.VMEM((NCHUNK, CHUNK), jnp.int32),
        pltpu.VMEM((NCHUNK, CHUNK), jnp.int32),
        pltpu.VMEM((BPW,), jnp.float32),
        pltpu.VMEM((BPW,), jnp.float32),
        pltpu.VMEM((BPW,), jnp.float32),
        pltpu.VMEM((2 * D + 3, L), jnp.float32),
        pltpu.SemaphoreType.DMA,
    ],
)
def _gather_kernel(user_hbm, skill_hbm, udot_hbm, sdot_hbm, wb_hbm, out_hbm,
                   ui, si, gu, gs, out_v, wv, sem):
    wid = lax.axis_index("s") * NC + lax.axis_index("c")
    base = wid * BPW

    pltpu.sync_copy(wb_hbm, wv)
    pltpu.sync_copy(user_hbm.at[wid], ui)
    pltpu.sync_copy(skill_hbm.at[wid], si)

    copies = []
    for j in range(NCHUNK):
        copies.append(pltpu.async_copy(
            udot_hbm.at[ui.at[j]], gu.at[pl.ds(j * CHUNK, CHUNK)], sem))
        copies.append(pltpu.async_copy(
            sdot_hbm.at[si.at[j]], gs.at[pl.ds(j * CHUNK, CHUNK)], sem))
    for c in copies:
        c.wait()

    bb = wv[2 * D]

    def groups(g, carry):
        out_v[pl.ds(g * L, L)] = (gu[pl.ds(g * L, L)] + gs[pl.ds(g * L, L)]
                                  + bb)
        return carry

    lax.fori_loop(0, BPW // L, groups, 0)
    pltpu.sync_copy(out_v, out_hbm.at[pl.ds(base, BPW)])


def kernel(user, skill, user_table, skill_table, W, b):
    user_r = user.astype(jnp.int32).reshape(NW, NCHUNK, CHUNK)
    skill_r = skill.astype(jnp.int32).reshape(NW, NCHUNK, CHUNK)
    u_tail = user_table[NFULL_U * S:].reshape((NU - NFULL_U * S) * D)
    s_tail = skill_table[NFULL_S * S:].reshape((NSK - NFULL_S * S) * D)
    wb = jnp.concatenate([
        jnp.broadcast_to(
            jnp.concatenate([W[:, 0], b.astype(jnp.float32)])[:, None],
            (2 * D + 1, L),
        ),
        W[:D, 0][None, :],
        W[D:, 0][None, :],
    ]).astype(jnp.float32)
    udot, sdot = _proj_kernel(user_table.T, skill_table.T, u_tail, s_tail, wb)
    return _gather_kernel(user_r, skill_r, udot, sdot, wb)


# speedup vs baseline: 1.6609x; 1.1024x over previous
import functools

import jax
import jax.numpy as jnp
from jax import lax
from jax.experimental import pallas as pl
from jax.experimental.pallas import tpu as pltpu
from jax.experimental.pallas import tpu_sc as plsc

B = 16384
D = 16
L = 16
NC = 2
NS = 16
NW = NC * NS
BPW = B // NW
NCHUNK = 4
CHUNK = BPW // NCHUNK

NU = 1000000
NSK = 100000
S = 2048
NFULL_U = NU // S
NFULL_S = NSK // S
TU = (NFULL_U + NW - 1) // NW
TS = (NFULL_S + NW - 1) // NW


def _xperm(x, s, lane):
    return jnp.take_along_axis(x, lane ^ s, axis=0, mode="promise_in_bounds")


def _sum16(ps, lane):
    def combine(x, y, s):
        return jnp.where((lane & s) == 0, x + _xperm(x, s, lane),
                         y + _xperm(y, s, lane))

    ps = [combine(ps[i], ps[i + 8], 8) for i in range(8)]
    ps = [combine(ps[i], ps[i + 4], 4) for i in range(4)]
    ps = [combine(ps[i], ps[i + 2], 2) for i in range(2)]
    return combine(ps[0], ps[1], 1)


def _phase_a(src, dst, w_rows, nfull, t_count, w, stas, ods, sem_st, sem_wb):
    def fire_stage(c, buf):
        pltpu.async_copy(src.at[:, pl.ds(c * S, S)], stas[buf],
                         sem_st.at[buf])

    def wait_stage(buf):
        pltpu.make_async_copy(src.at[:, pl.ds(0, S)],
                              stas[buf], sem_st.at[buf]).wait()

    def wait_wb(buf):
        pltpu.make_async_copy(ods[0], dst.at[pl.ds(0, S)],
                              sem_wb.at[buf]).wait()

    @pl.when(w < nfull)
    def _():
        fire_stage(w, 0)

    def process(t, buf):
        c = w + NW * t

        @pl.when(c < nfull)
        def _():
            wait_stage(buf)
            nxt = c + NW

            @pl.when(nxt < nfull)
            def _():
                fire_stage(nxt, 1 - buf)

            @pl.when(t >= 2)
            def _():
                wait_wb(buf)

            @plsc.parallel_loop(0, S // L, unroll=2)
            def groups(g):
                vs = [stas[buf][k, pl.ds(g * L, L)] * w_rows[k]
                      for k in range(D)]
                while len(vs) > 1:
                    vs = [vs[i] + vs[i + 1] for i in range(0, len(vs), 2)]
                ods[buf][pl.ds(g * L, L)] = vs[0]
            pltpu.async_copy(ods[buf], dst.at[pl.ds(c * S, S)],
                             sem_wb.at[buf])
        return None

    def body(th, carry):
        process(2 * th, 0)
        process(2 * th + 1, 1)
        return carry

    lax.fori_loop(0, (t_count + 1) // 2, body, 0)

    nt = jnp.where(w < nfull, (nfull - 1 - w) // NW + 1, 0)

    @pl.when(nt >= 2)
    def _():
        wait_wb(nt & 1)

    @pl.when(nt >= 1)
    def _():
        wait_wb((nt + 1) & 1)


def _tail_dots(tail, dst, row0, n, w_vec, tb, od, lane):
    pltpu.sync_copy(tail, tb.at[pl.ds(0, n * D)])

    def group(g, carry):
        ps = [tb[pl.ds((g * L + j) * D, D)] * w_vec for j in range(L)]
        od[pl.ds(g * L, L)] = _sum16(ps, lane)
        return carry

    lax.fori_loop(0, n // L, group, 0)
    pltpu.sync_copy(od.at[pl.ds(0, n)], dst.at[pl.ds(row0, n)])


@functools.partial(
    pl.kernel,
    out_type=(jax.ShapeDtypeStruct((NU,), jnp.float32),
              jax.ShapeDtypeStruct((NSK,), jnp.float32)),
    mesh=plsc.VectorSubcoreMesh(core_axis_name="c", subcore_axis_name="s"),
    scratch_types=[
        pltpu.VMEM((D, S), jnp.float32),
        pltpu.VMEM((D, S), jnp.float32),
        pltpu.VMEM((S,), jnp.float32),
        pltpu.VMEM((S,), jnp.float32),
        pltpu.VMEM(((NSK - NFULL_S * S) * D,), jnp.float32),
        pltpu.VMEM((2 * D + 3, L), jnp.float32),
        pltpu.SemaphoreType.DMA((2,)),
        pltpu.SemaphoreType.DMA((2,)),
    ],
)
def _proj_kernel(ut_t, st_t, u_tail, s_tail, wb_hbm, udot, sdot,
                 sta0, sta1, od0, od1, tb, wv, sem_st, sem_wb):
    w = lax.axis_index("s") * NC + lax.axis_index("c")
    lane = lax.iota(jnp.int32, L)
    pltpu.sync_copy(wb_hbm, wv)
    wu_rows = [wv[k] for k in range(D)]
    ws_rows = [wv[D + k] for k in range(D)]
    stas, ods = (sta0, sta1), (od0, od1)
    _phase_a(ut_t, udot, wu_rows, NFULL_U, TU, w, stas, ods, sem_st, sem_wb)
    _phase_a(st_t, sdot, ws_rows, NFULL_S, TS, w, stas, ods, sem_st, sem_wb)

    @pl.when(w == 0)
    def _():
        _tail_dots(u_tail, udot, NFULL_U * S, NU - NFULL_U * S,
                   wv[2 * D + 1], tb, od0, lane)

    @pl.when(w == 1)
    def _():
        _tail_dots(s_tail, sdot, NFULL_S * S, NSK - NFULL_S * S,
                   wv[2 * D + 2], tb, od0, lane)


@functools.partial(
    pl.kernel,
    out_type=jax.ShapeDtypeStruct((B,), jnp.float32),
    mesh=plsc.VectorSubcoreMesh(core_axis_name="c", subcore_axis_name="s"),
    scratch_types=[
        pltpu.VMEM((NCHUNK, CHUNK), jnp.int32),
        pltpu.VMEM((NCHUNK, CHUNK), jnp.int32),
        pltpu.VMEM((BPW,), jnp.float32),
        pltpu.VMEM((BPW,), jnp.float32),
        pltpu.VMEM((BPW,), jnp.float32),
        pltpu.VMEM((2 * D + 3, L), jnp.float32),
        pltpu.SemaphoreType.DMA,
    ],
)
def _gather_kernel(user_hbm, skill_hbm, udot_hbm, sdot_hbm, wb_hbm, out_hbm,
                   ui, si, gu, gs, out_v, wv, sem):
    wid = lax.axis_index("s") * NC + lax.axis_index("c")
    base = wid * BPW

    pltpu.sync_copy(wb_hbm, wv)
    pltpu.sync_copy(user_hbm.at[wid], ui)
    pltpu.sync_copy(skill_hbm.at[wid], si)

    copies = []
    for j in range(NCHUNK):
        copies.append(pltpu.async_copy(
            udot_hbm.at[ui.at[j]], gu.at[pl.ds(j * CHUNK, CHUNK)], sem))
        copies.append(pltpu.async_copy(
            sdot_hbm.at[si.at[j]], gs.at[pl.ds(j * CHUNK, CHUNK)], sem))
    for c in copies:
        c.wait()

    bb = wv[2 * D]

    def groups(g, carry):
        out_v[pl.ds(g * L, L)] = (gu[pl.ds(g * L, L)] + gs[pl.ds(g * L, L)]
                                  + bb)
        return carry

    lax.fori_loop(0, BPW // L, groups, 0)
    pltpu.sync_copy(out_v, out_hbm.at[pl.ds(base, BPW)])


def kernel(user, skill, user_table, skill_table, W, b):
    user_r = user.astype(jnp.int32).reshape(NW, NCHUNK, CHUNK)
    skill_r = skill.astype(jnp.int32).reshape(NW, NCHUNK, CHUNK)
    u_tail = user_table[NFULL_U * S:].reshape((NU - NFULL_U * S) * D)
    s_tail = skill_table[NFULL_S * S:].reshape((NSK - NFULL_S * S) * D)
    wb = jnp.concatenate([
        jnp.broadcast_to(
            jnp.concatenate([W[:, 0], b.astype(jnp.float32)])[:, None],
            (2 * D + 1, L),
        ),
        W[:D, 0][None, :],
        W[D:, 0][None, :],
    ]).astype(jnp.float32)
    udot, sdot = _proj_kernel(user_table.T, skill_table.T, u_tail, s_tail, wb)
    return _gather_kernel(user_r, skill_r, udot, sdot, wb)
